# baseline (speedup 1.0000x reference)
import functools

import jax
import jax.numpy as jnp
from jax import lax
from jax.experimental import pallas as pl
from jax.experimental.pallas import tpu as pltpu
from jax.experimental.pallas import tpu_sc as plsc

N = 10000
E = 320000
D_IN = 128
D_H = 64

NC = 2
NS = 16
L = 16
NW = NC * NS

K = 128
T_CH = 2 * (-(-E // (NW * K * 2)))
E_PAD = NW * T_CH * K
N_PAD = 10240
HR = N_PAD // L
RPT = N_PAD // NS


@functools.cache
def _sc_degree_kernel():
    mesh = plsc.VectorSubcoreMesh(core_axis_name="c", subcore_axis_name="s")
    return pl.kernel(
        _sc_degree_body,
        out_type=jax.ShapeDtypeStruct((NC, N_PAD), jnp.float32),
        mesh=mesh,
        scratch_types=[
            pltpu.VMEM((T_CH, K), jnp.int32),
            pltpu.VMEM((N_PAD,), jnp.float32),
            pltpu.VMEM((N_PAD // K, K), jnp.int32),
            pltpu.VMEM_SHARED((N_PAD,), jnp.float32),
        ],
        compiler_params=pltpu.CompilerParams(needs_layout_passes=False),
    )


def _sc_degree_body(dst_hbm, out_hbm, dst_v, hist_v, row_idx, acc):
    c = lax.axis_index("c")
    s = lax.axis_index("s")
    w = c * NS + s
    zeros16 = jnp.zeros((L,), jnp.float32)
    ones16 = jnp.full((L,), 1.0, jnp.float32)
    iota16 = lax.iota(jnp.int32, L)

    def zero_row(i, _):
        hist_v[pl.ds(i * L, L)] = zeros16
        return 0
    lax.fori_loop(0, HR, zero_row, 0)

    pltpu.sync_copy(hist_v.at[pl.ds(0, N_PAD // NS)],
                    acc.at[pl.ds(s * (N_PAD // NS), N_PAD // NS)])

    def ri(j, _):
        def rk(kk, _):
            row_idx[j, pl.ds(kk * L, L)] = j * K + kk * L + iota16
            return 0
        return lax.fori_loop(0, K // L, rk, 0)
    lax.fori_loop(0, N_PAD // K, ri, 0)

    plsc.subcore_barrier()

    pltpu.sync_copy(dst_hbm.at[w], dst_v)

    def body(j, _):
        def inner(kk, _):
            idx = dst_v[j, pl.ds(kk * L, L)]
            plsc.addupdate_scatter(hist_v, [idx], ones16)
            return 0
        return lax.fori_loop(0, K // L, inner, 0)
    lax.fori_loop(0, T_CH, body, 0)

    def comb(j, _):
        pltpu.sync_copy(hist_v.at[pl.ds(j * K, K)],
                        acc.at[row_idx.at[j]], add=True)
        return 0
    lax.fori_loop(0, N_PAD // K, comb, 0)

    plsc.subcore_barrier()
    pltpu.sync_copy(acc.at[pl.ds(s * (N_PAD // NS), N_PAD // NS)],
                    out_hbm.at[c, pl.ds(s * (N_PAD // NS), N_PAD // NS)])


@functools.cache
def _sc_scatter_kernel():
    mesh = plsc.VectorSubcoreMesh(core_axis_name="c", subcore_axis_name="s")
    return pl.kernel(
        _sc_scatter_body,
        out_type=jax.ShapeDtypeStruct((NC, N_PAD, D_H), jnp.float32),
        mesh=mesh,
        scratch_types=[
            pltpu.VMEM((T_CH, K), jnp.int32),
            pltpu.VMEM((T_CH, K), jnp.int32),
            pltpu.VMEM((K, D_H), jnp.float32),
            pltpu.VMEM((K, D_H), jnp.float32),
            pltpu.VMEM((64, D_H), jnp.float32),
            pltpu.VMEM_SHARED((N_PAD, D_H), jnp.float32),
            pltpu.SemaphoreType.DMA,
            pltpu.SemaphoreType.DMA,
        ],
        compiler_params=pltpu.CompilerParams(
            needs_layout_passes=False, use_tc_tiling_on_sc=False),
    )


def _sc_scatter_body(g_hbm, src_hbm, dst_hbm, out_hbm,
                     src_v, dst_v, rows0, rows1, zbuf, acc, sem0, sem1):
    c = lax.axis_index("c")
    s = lax.axis_index("s")
    w = c * NS + s
    zeros16 = jnp.zeros((L,), jnp.float32)

    def zrow(i, _):
        def zcol(kk, _):
            zbuf[i, pl.ds(kk * L, L)] = zeros16
            return 0
        return lax.fori_loop(0, D_H // L, zcol, 0)
    lax.fori_loop(0, 64, zrow, 0)

    def zc(i, _):
        pltpu.sync_copy(zbuf, acc.at[pl.ds(s * RPT + i * 64, 64)])
        return 0
    lax.fori_loop(0, RPT // 64, zc, 0)

    plsc.subcore_barrier()

    pltpu.sync_copy(src_hbm.at[w], src_v)
    pltpu.sync_copy(dst_hbm.at[w], dst_v)

    def start(j, rows, sem):
        pltpu.async_copy(g_hbm.at[src_v.at[j]], rows, sem)

    def finish(j, rows, sem):
        pltpu.make_async_copy(g_hbm.at[src_v.at[j]], rows, sem).wait()
        pltpu.sync_copy(rows, acc.at[dst_v.at[j]], add=True)

    start(0, rows0, sem0)
    start(1, rows1, sem1)

    def body(jj, _):
        j0 = 2 * jj
        finish(j0, rows0, sem0)
        start(j0 + 2, rows0, sem0)
        finish(j0 + 1, rows1, sem1)
        start(j0 + 3, rows1, sem1)
        return 0
    lax.fori_loop(0, (T_CH - 2) // 2, body, 0)
    finish(T_CH - 2, rows0, sem0)
    finish(T_CH - 1, rows1, sem1)

    plsc.subcore_barrier()
    pltpu.sync_copy(acc.at[pl.ds(s * RPT, RPT)],
                    out_hbm.at[c, pl.ds(s * RPT, RPT)])


_R = 512
_G = N_PAD // _R


def _tc1_body(x_ref, w_ref, d0_ref, d1_ref, g_ref, dis_ref):
    deg = d0_ref[...] + d1_ref[...] + 1.0
    dis = lax.rsqrt(deg)
    h = jnp.dot(x_ref[...], w_ref[...], preferred_element_type=jnp.float32)
    g_ref[...] = h * dis
    dis_ref[...] = dis


def _tc1(x_pad, W1, d0, d1):
    return pl.pallas_call(
        _tc1_body,
        grid=(_G,),
        in_specs=[
            pl.BlockSpec((_R, D_IN), lambda m: (m, 0)),
            pl.BlockSpec((D_IN, D_H), lambda m: (0, 0)),
            pl.BlockSpec((_R, 1), lambda m: (m, 0)),
            pl.BlockSpec((_R, 1), lambda m: (m, 0)),
        ],
        out_specs=[
            pl.BlockSpec((_R, D_H), lambda m: (m, 0)),
            pl.BlockSpec((_R, 1), lambda m: (m, 0)),
        ],
        out_shape=[
            jax.ShapeDtypeStruct((N_PAD, D_H), jnp.float32),
            jax.ShapeDtypeStruct((N_PAD, 1), jnp.float32),
        ],
    )(x_pad, W1, d0, d1)


def _tc2_body(p0_ref, p1_ref, g1_ref, dis_ref, b1_ref, w2_ref, g2_ref):
    dis = dis_ref[...]
    agg = p0_ref[...] + p1_ref[...] + g1_ref[...]
    h = jnp.maximum(agg * dis + b1_ref[0:1, :], 0.0)
    g2_ref[...] = jnp.dot(h, w2_ref[...],
                          preferred_element_type=jnp.float32) * dis


def _tc2(p0, p1, g1, dis, b1r, W2):
    return pl.pallas_call(
        _tc2_body,
        grid=(_G,),
        in_specs=[
            pl.BlockSpec((_R, D_H), lambda m: (m, 0)),
            pl.BlockSpec((_R, D_H), lambda m: (m, 0)),
            pl.BlockSpec((_R, D_H), lambda m: (m, 0)),
            pl.BlockSpec((_R, 1), lambda m: (m, 0)),
            pl.BlockSpec((8, D_H), lambda m: (0, 0)),
            pl.BlockSpec((D_H, D_H), lambda m: (0, 0)),
        ],
        out_specs=pl.BlockSpec((_R, D_H), lambda m: (m, 0)),
        out_shape=jax.ShapeDtypeStruct((N_PAD, D_H), jnp.float32),
    )(p0, p1, g1, dis, b1r, W2)


def _tc3_body(p0_ref, p1_ref, g2_ref, dis_ref, b2_ref, wc_ref, bc_ref, y_ref):
    dis = dis_ref[...]
    agg = p0_ref[...] + p1_ref[...] + g2_ref[...]
    h = jnp.maximum(agg * dis + b2_ref[0:1, :], 0.0)
    z = jnp.dot(h, wc_ref[...],
                preferred_element_type=jnp.float32) + bc_ref[0:1, 0:1]
    y_ref[...] = jax.nn.sigmoid(z)


def _tc3(p0, p1, g2, dis, b2r, Wc, bcr):
    return pl.pallas_call(
        _tc3_body,
        grid=(_G,),
        in_specs=[
            pl.BlockSpec((_R, D_H), lambda m: (m, 0)),
            pl.BlockSpec((_R, D_H), lambda m: (m, 0)),
            pl.BlockSpec((_R, D_H), lambda m: (m, 0)),
            pl.BlockSpec((_R, 1), lambda m: (m, 0)),
            pl.BlockSpec((8, D_H), lambda m: (0, 0)),
            pl.BlockSpec((D_H, 1), lambda m: (0, 0)),
            pl.BlockSpec((8, 1), lambda m: (0, 0)),
        ],
        out_specs=pl.BlockSpec((_R, 1), lambda m: (m, 0)),
        out_shape=jax.ShapeDtypeStruct((N_PAD, 1), jnp.float32),
    )(p0, p1, g2, dis, b2r, Wc, bcr)


def kernel(x, edge_index, W1, b1, W2, b2, Wc, bc):
    src = edge_index[0].astype(jnp.int32)
    dst = edge_index[1].astype(jnp.int32)
    src3 = jnp.pad(src, (0, E_PAD - E)).reshape(NW, T_CH, K)
    dst3 = jnp.pad(dst, (0, E_PAD - E),
                   constant_values=N).reshape(NW, T_CH, K)

    deg = _sc_degree_kernel()(dst3)
    d = deg.reshape(NC, N_PAD, 1)

    x_pad = jnp.pad(x, ((0, N_PAD - N), (0, 0)))
    g1, dis = _tc1(x_pad, W1, d[0], d[1])

    b1r = jnp.broadcast_to(b1[None, :], (8, D_H))
    b2r = jnp.broadcast_to(b2[None, :], (8, D_H))
    bcr = jnp.broadcast_to(bc[None, :], (8, 1))

    p = _sc_scatter_kernel()(g1, src3, dst3)
    g2 = _tc2(p[0], p[1], g1, dis, b1r, W2)

    q = _sc_scatter_kernel()(g2, src3, dst3)
    y = _tc3(q[0], q[1], g2, dis, b2r, Wc, bcr)
    return y[:N]


# speedup vs baseline: 1.6380x; 1.6380x over previous
import functools

import jax
import jax.numpy as jnp
from jax import lax
from jax.experimental import pallas as pl
from jax.experimental.pallas import tpu as pltpu
from jax.experimental.pallas import tpu_sc as plsc

N = 10000
E = 320000
D_IN = 128
D_H = 64

NC = 2
NS = 16
L = 16
NW = NC * NS

EPW = E // NW
K = 125
T_CH = EPW // K
N_PAD = 10240
HR = N_PAD // L
RPT = N_PAD // NS
CK = 128


@functools.cache
def _sc_degree_kernel():
    mesh = plsc.VectorSubcoreMesh(core_axis_name="c", subcore_axis_name="s")
    return pl.kernel(
        _sc_degree_body,
        out_type=jax.ShapeDtypeStruct((NC, N_PAD), jnp.float32),
        mesh=mesh,
        scratch_types=[
            pltpu.VMEM((EPW,), jnp.int32),
            pltpu.VMEM((N_PAD,), jnp.float32),
            pltpu.VMEM((N_PAD // CK, CK), jnp.int32),
            pltpu.VMEM_SHARED((N_PAD,), jnp.float32),
        ],
        compiler_params=pltpu.CompilerParams(needs_layout_passes=False),
    )


def _sc_degree_body(dst_hbm, out_hbm, dst_v, hist_v, row_idx, acc):
    c = lax.axis_index("c")
    s = lax.axis_index("s")
    w = c * NS + s
    zeros16 = jnp.zeros((L,), jnp.float32)
    ones16 = jnp.full((L,), 1.0, jnp.float32)
    iota16 = lax.iota(jnp.int32, L)

    def zero_row(i, _):
        hist_v[pl.ds(i * L, L)] = zeros16
        return 0
    lax.fori_loop(0, HR, zero_row, 0)

    pltpu.sync_copy(hist_v.at[pl.ds(0, N_PAD // NS)],
                    acc.at[pl.ds(s * (N_PAD // NS), N_PAD // NS)])

    def ri(j, _):
        def rk(kk, _):
            row_idx[j, pl.ds(kk * L, L)] = j * CK + kk * L + iota16
            return 0
        return lax.fori_loop(0, CK // L, rk, 0)
    lax.fori_loop(0, N_PAD // CK, ri, 0)

    plsc.subcore_barrier()

    pltpu.sync_copy(dst_hbm.at[w], dst_v)

    def body(i, _):
        idx = dst_v[pl.ds(i * L, L)]
        plsc.addupdate_scatter(hist_v, [idx], ones16)
        return 0
    lax.fori_loop(0, EPW // L, body, 0)

    def comb(j, _):
        pltpu.sync_copy(hist_v.at[pl.ds(j * CK, CK)],
                        acc.at[row_idx.at[j]], add=True)
        return 0
    lax.fori_loop(0, N_PAD // CK, comb, 0)

    plsc.subcore_barrier()
    pltpu.sync_copy(acc.at[pl.ds(s * (N_PAD // NS), N_PAD // NS)],
                    out_hbm.at[c, pl.ds(s * (N_PAD // NS), N_PAD // NS)])


@functools.cache
def _sc_scatter_kernel():
    mesh = plsc.VectorSubcoreMesh(core_axis_name="c", subcore_axis_name="s")
    return pl.kernel(
        _sc_scatter_body,
        out_type=jax.ShapeDtypeStruct((NC, N_PAD, D_H), jnp.float32),
        mesh=mesh,
        scratch_types=[
            pltpu.VMEM((T_CH, K), jnp.int32),
            pltpu.VMEM((T_CH, K), jnp.int32),
            pltpu.VMEM((K, D_H), jnp.float32),
            pltpu.VMEM((64, D_H), jnp.float32),
            pltpu.VMEM_SHARED((N_PAD, D_H), jnp.float32),
            pltpu.SemaphoreType.DMA,
        ],
        compiler_params=pltpu.CompilerParams(
            needs_layout_passes=False, use_tc_tiling_on_sc=False),
    )


def _sc_scatter_body(g_hbm, src_hbm, dst_hbm, out_hbm,
                     src_v, dst_v, rows_v, zbuf, acc, sem):
    c = lax.axis_index("c")
    s = lax.axis_index("s")
    w = c * NS + s
    zeros16 = jnp.zeros((L,), jnp.float32)

    def zrow(i, _):
        def zcol(kk, _):
            zbuf[i, pl.ds(kk * L, L)] = zeros16
            return 0
        return lax.fori_loop(0, D_H // L, zcol, 0)
    lax.fori_loop(0, 64, zrow, 0)

    def zc(i, _):
        pltpu.sync_copy(zbuf, acc.at[pl.ds(s * RPT + i * 64, 64)])
        return 0
    lax.fori_loop(0, RPT // 64, zc, 0)

    plsc.subcore_barrier()

    pltpu.sync_copy(src_hbm.at[w], src_v)
    pltpu.sync_copy(dst_hbm.at[w], dst_v)

    def body(j, _):
        pltpu.async_copy(g_hbm.at[src_v.at[j]], rows_v, sem).wait()
        pltpu.sync_copy(rows_v, acc.at[dst_v.at[j]], add=True)
        return 0
    lax.fori_loop(0, T_CH, body, 0)

    plsc.subcore_barrier()
    pltpu.sync_copy(acc.at[pl.ds(s * RPT, RPT)],
                    out_hbm.at[c, pl.ds(s * RPT, RPT)])


_R = 400
_G = N // _R


def _tc1_body(x_ref, w_ref, d0_ref, d1_ref, g_ref, dis_ref):
    deg = d0_ref[...] + d1_ref[...] + 1.0
    dis = lax.rsqrt(deg)
    h = jnp.dot(x_ref[...], w_ref[...], preferred_element_type=jnp.float32)
    g_ref[...] = h * dis
    dis_ref[...] = dis


def _tc1(x, W1, d0, d1):
    return pl.pallas_call(
        _tc1_body,
        grid=(_G,),
        in_specs=[
            pl.BlockSpec((_R, D_IN), lambda m: (m, 0)),
            pl.BlockSpec((D_IN, D_H), lambda m: (0, 0)),
            pl.BlockSpec((_R, 1), lambda m: (m, 0)),
            pl.BlockSpec((_R, 1), lambda m: (m, 0)),
        ],
        out_specs=[
            pl.BlockSpec((_R, D_H), lambda m: (m, 0)),
            pl.BlockSpec((_R, 1), lambda m: (m, 0)),
        ],
        out_shape=[
            jax.ShapeDtypeStruct((N, D_H), jnp.float32),
            jax.ShapeDtypeStruct((N, 1), jnp.float32),
        ],
    )(x, W1, d0, d1)


def _tc2_body(p0_ref, p1_ref, g1_ref, dis_ref, b1_ref, w2_ref, g2_ref):
    dis = dis_ref[...]
    agg = p0_ref[...] + p1_ref[...] + g1_ref[...]
    h = jnp.maximum(agg * dis + b1_ref[0:1, :], 0.0)
    g2_ref[...] = jnp.dot(h, w2_ref[...],
                          preferred_element_type=jnp.float32) * dis


def _tc2(p0, p1, g1, dis, b1r, W2):
    return pl.pallas_call(
        _tc2_body,
        grid=(_G,),
        in_specs=[
            pl.BlockSpec((_R, D_H), lambda m: (m, 0)),
            pl.BlockSpec((_R, D_H), lambda m: (m, 0)),
            pl.BlockSpec((_R, D_H), lambda m: (m, 0)),
            pl.BlockSpec((_R, 1), lambda m: (m, 0)),
            pl.BlockSpec((8, D_H), lambda m: (0, 0)),
            pl.BlockSpec((D_H, D_H), lambda m: (0, 0)),
        ],
        out_specs=pl.BlockSpec((_R, D_H), lambda m: (m, 0)),
        out_shape=jax.ShapeDtypeStruct((N, D_H), jnp.float32),
    )(p0, p1, g1, dis, b1r, W2)


def _tc3_body(p0_ref, p1_ref, g2_ref, dis_ref, b2_ref, wc_ref, bc_ref, y_ref):
    dis = dis_ref[...]
    agg = p0_ref[...] + p1_ref[...] + g2_ref[...]
    h = jnp.maximum(agg * dis + b2_ref[0:1, :], 0.0)
    z = jnp.dot(h, wc_ref[...],
                preferred_element_type=jnp.float32) + bc_ref[0:1, 0:1]
    y_ref[...] = jax.nn.sigmoid(z)


def _tc3(p0, p1, g2, dis, b2r, Wc, bcr):
    return pl.pallas_call(
        _tc3_body,
        grid=(_G,),
        in_specs=[
            pl.BlockSpec((_R, D_H), lambda m: (m, 0)),
            pl.BlockSpec((_R, D_H), lambda m: (m, 0)),
            pl.BlockSpec((_R, D_H), lambda m: (m, 0)),
            pl.BlockSpec((_R, 1), lambda m: (m, 0)),
            pl.BlockSpec((8, D_H), lambda m: (0, 0)),
            pl.BlockSpec((D_H, 1), lambda m: (0, 0)),
            pl.BlockSpec((8, 1), lambda m: (0, 0)),
        ],
        out_specs=pl.BlockSpec((_R, 1), lambda m: (m, 0)),
        out_shape=jax.ShapeDtypeStruct((N, 1), jnp.float32),
    )(p0, p1, g2, dis, b2r, Wc, bcr)


def kernel(x, edge_index, W1, b1, W2, b2, Wc, bc):
    src = edge_index[0].astype(jnp.int32)
    dst = edge_index[1].astype(jnp.int32)
    src3 = src.reshape(NW, T_CH, K)
    dst3 = dst.reshape(NW, T_CH, K)
    dst2 = dst.reshape(NW, EPW)

    deg = _sc_degree_kernel()(dst2)
    d = deg.reshape(NC, N_PAD, 1)

    g1, dis = _tc1(x, W1, d[0], d[1])

    b1r = jnp.broadcast_to(b1[None, :], (8, D_H))
    b2r = jnp.broadcast_to(b2[None, :], (8, D_H))
    bcr = jnp.broadcast_to(bc[None, :], (8, 1))

    p = _sc_scatter_kernel()(g1, src3, dst3)
    g2 = _tc2(p[0], p[1], g1, dis, b1r, W2)

    q = _sc_scatter_kernel()(g2, src3, dst3)
    y = _tc3(q[0], q[1], g2, dis, b2r, Wc, bcr)
    return y


# speedup vs baseline: 1.8910x; 1.1544x over previous
import functools

import jax
import jax.numpy as jnp
from jax import lax
from jax.experimental import pallas as pl
from jax.experimental.pallas import tpu as pltpu
from jax.experimental.pallas import tpu_sc as plsc

N = 10000
E = 320000
D_IN = 128
D_H = 64

NC = 2
NS = 16
L = 16
NW = NC * NS

EPW = E // NW
K = 128
T_CH = EPW // K
TAIL = EPW - T_CH * K
N_PAD = 10240
HR = N_PAD // L
RPT = N_PAD // NS
CK = 128


@functools.cache
def _sc_degree_kernel():
    mesh = plsc.VectorSubcoreMesh(core_axis_name="c", subcore_axis_name="s")
    return pl.kernel(
        _sc_degree_body,
        out_type=jax.ShapeDtypeStruct((NC, N_PAD), jnp.float32),
        mesh=mesh,
        scratch_types=[
            pltpu.VMEM((EPW,), jnp.int32),
            pltpu.VMEM((N_PAD,), jnp.float32),
            pltpu.VMEM((N_PAD // CK, CK), jnp.int32),
            pltpu.VMEM_SHARED((N_PAD,), jnp.float32),
        ],
        compiler_params=pltpu.CompilerParams(
            needs_layout_passes=False, use_tc_tiling_on_sc=False),
    )


def _sc_degree_body(ei_hbm, out_hbm, dst_v, hist_v, row_idx, acc):
    c = lax.axis_index("c")
    s = lax.axis_index("s")
    w = c * NS + s
    zeros16 = jnp.zeros((L,), jnp.float32)
    ones16 = jnp.full((L,), 1.0, jnp.float32)
    iota16 = lax.iota(jnp.int32, L)

    def zero_row(i, _):
        hist_v[pl.ds(i * L, L)] = zeros16
        return 0
    lax.fori_loop(0, HR, zero_row, 0)

    pltpu.sync_copy(hist_v.at[pl.ds(0, N_PAD // NS)],
                    acc.at[pl.ds(s * (N_PAD // NS), N_PAD // NS)])

    def ri(j, _):
        def rk(kk, _):
            row_idx[j, pl.ds(kk * L, L)] = j * CK + kk * L + iota16
            return 0
        return lax.fori_loop(0, CK // L, rk, 0)
    lax.fori_loop(0, N_PAD // CK, ri, 0)

    plsc.subcore_barrier()

    pltpu.sync_copy(ei_hbm.at[1, pl.ds(w * EPW, EPW)], dst_v)

    def body(i, _):
        idx = dst_v[pl.ds(i * L, L)]
        plsc.addupdate_scatter(hist_v, [idx], ones16)
        return 0
    lax.fori_loop(0, EPW // L, body, 0)

    def comb(j, _):
        pltpu.sync_copy(hist_v.at[pl.ds(j * CK, CK)],
                        acc.at[row_idx.at[j]], add=True)
        return 0
    lax.fori_loop(0, N_PAD // CK, comb, 0)

    plsc.subcore_barrier()
    pltpu.sync_copy(acc.at[pl.ds(s * (N_PAD // NS), N_PAD // NS)],
                    out_hbm.at[c, pl.ds(s * (N_PAD // NS), N_PAD // NS)])


@functools.cache
def _sc_scatter_kernel():
    mesh = plsc.VectorSubcoreMesh(core_axis_name="c", subcore_axis_name="s")
    return pl.kernel(
        _sc_scatter_body,
        out_type=jax.ShapeDtypeStruct((NC, N_PAD, D_H), jnp.float32),
        mesh=mesh,
        scratch_types=[
            pltpu.VMEM((EPW,), jnp.int32),
            pltpu.VMEM((EPW,), jnp.int32),
            pltpu.VMEM((K, D_H), jnp.float32),
            pltpu.VMEM((64, D_H), jnp.float32),
            pltpu.VMEM_SHARED((N_PAD, D_H), jnp.float32),
            pltpu.SemaphoreType.DMA,
        ],
        compiler_params=pltpu.CompilerParams(
            needs_layout_passes=False, use_tc_tiling_on_sc=False),
    )


def _sc_scatter_body(g_hbm, ei_hbm, out_hbm,
                     src_v, dst_v, rows_v, zbuf, acc, sem):
    c = lax.axis_index("c")
    s = lax.axis_index("s")
    w = c * NS + s
    zeros16 = jnp.zeros((L,), jnp.float32)

    def zrow(i, _):
        def zcol(kk, _):
            zbuf[i, pl.ds(kk * L, L)] = zeros16
            return 0
        return lax.fori_loop(0, D_H // L, zcol, 0)
    lax.fori_loop(0, 64, zrow, 0)

    def zc(i, _):
        pltpu.sync_copy(zbuf, acc.at[pl.ds(s * RPT + i * 64, 64)])
        return 0
    lax.fori_loop(0, RPT // 64, zc, 0)

    plsc.subcore_barrier()

    pltpu.sync_copy(ei_hbm.at[0, pl.ds(w * EPW, EPW)], src_v)
    pltpu.sync_copy(ei_hbm.at[1, pl.ds(w * EPW, EPW)], dst_v)

    def body(j, _):
        pltpu.async_copy(g_hbm.at[src_v.at[pl.ds(j * K, K)]],
                         rows_v, sem).wait()
        pltpu.sync_copy(rows_v, acc.at[dst_v.at[pl.ds(j * K, K)]], add=True)
        return 0
    lax.fori_loop(0, T_CH, body, 0)

    pltpu.async_copy(g_hbm.at[src_v.at[pl.ds(T_CH * K, TAIL)]],
                     rows_v.at[pl.ds(0, TAIL)], sem).wait()
    pltpu.sync_copy(rows_v.at[pl.ds(0, TAIL)],
                    acc.at[dst_v.at[pl.ds(T_CH * K, TAIL)]], add=True)

    plsc.subcore_barrier()
    pltpu.sync_copy(acc.at[pl.ds(s * RPT, RPT)],
                    out_hbm.at[c, pl.ds(s * RPT, RPT)])


_R = 2000
_G = N // _R


def _tc1_body(x_ref, w_ref, d_ref, g_ref, dis_ref):
    deg = d_ref[0] + d_ref[1] + 1.0
    dis = lax.rsqrt(deg)
    h = jnp.dot(x_ref[...], w_ref[...], preferred_element_type=jnp.float32)
    g_ref[...] = h * dis
    dis_ref[...] = dis


def _tc1(x, W1, d3):
    return pl.pallas_call(
        _tc1_body,
        grid=(_G,),
        in_specs=[
            pl.BlockSpec((_R, D_IN), lambda m: (m, 0)),
            pl.BlockSpec((D_IN, D_H), lambda m: (0, 0)),
            pl.BlockSpec((NC, _R, 1), lambda m: (0, m, 0)),
        ],
        out_specs=[
            pl.BlockSpec((_R, D_H), lambda m: (m, 0)),
            pl.BlockSpec((_R, 1), lambda m: (m, 0)),
        ],
        out_shape=[
            jax.ShapeDtypeStruct((N, D_H), jnp.float32),
            jax.ShapeDtypeStruct((N, 1), jnp.float32),
        ],
    )(x, W1, d3)


def _tc2_body(p_ref, g1_ref, dis_ref, b1_ref, w2_ref, g2_ref):
    dis = dis_ref[...]
    agg = p_ref[0] + p_ref[1] + g1_ref[...]
    h = jnp.maximum(agg * dis + b1_ref[0:1, :], 0.0)
    g2_ref[...] = jnp.dot(h, w2_ref[...],
                          preferred_element_type=jnp.float32) * dis


def _tc2(p, g1, dis, b1r, W2):
    return pl.pallas_call(
        _tc2_body,
        grid=(_G,),
        in_specs=[
            pl.BlockSpec((NC, _R, D_H), lambda m: (0, m, 0)),
            pl.BlockSpec((_R, D_H), lambda m: (m, 0)),
            pl.BlockSpec((_R, 1), lambda m: (m, 0)),
            pl.BlockSpec((8, D_H), lambda m: (0, 0)),
            pl.BlockSpec((D_H, D_H), lambda m: (0, 0)),
        ],
        out_specs=pl.BlockSpec((_R, D_H), lambda m: (m, 0)),
        out_shape=jax.ShapeDtypeStruct((N, D_H), jnp.float32),
    )(p, g1, dis, b1r, W2)


def _tc3_body(p_ref, g2_ref, dis_ref, b2_ref, wc_ref, bc_ref, y_ref):
    dis = dis_ref[...]
    agg = p_ref[0] + p_ref[1] + g2_ref[...]
    h = jnp.maximum(agg * dis + b2_ref[0:1, :], 0.0)
    z = jnp.dot(h, wc_ref[...],
                preferred_element_type=jnp.float32) + bc_ref[0:1, 0:1]
    y_ref[...] = jax.nn.sigmoid(z)


def _tc3(p, g2, dis, b2r, Wc, bcr):
    return pl.pallas_call(
        _tc3_body,
        grid=(_G,),
        in_specs=[
            pl.BlockSpec((NC, _R, D_H), lambda m: (0, m, 0)),
            pl.BlockSpec((_R, D_H), lambda m: (m, 0)),
            pl.BlockSpec((_R, 1), lambda m: (m, 0)),
            pl.BlockSpec((8, D_H), lambda m: (0, 0)),
            pl.BlockSpec((D_H, 1), lambda m: (0, 0)),
            pl.BlockSpec((8, 1), lambda m: (0, 0)),
        ],
        out_specs=pl.BlockSpec((_R, 1), lambda m: (m, 0)),
        out_shape=jax.ShapeDtypeStruct((N, 1), jnp.float32),
    )(p, g2, dis, b2r, Wc, bcr)


def kernel(x, edge_index, W1, b1, W2, b2, Wc, bc):
    ei = edge_index.astype(jnp.int32)

    deg = _sc_degree_kernel()(ei)
    d3 = deg.reshape(NC, N_PAD, 1)

    g1, dis = _tc1(x, W1, d3)

    b1r = jnp.broadcast_to(b1[None, :], (8, D_H))
    b2r = jnp.broadcast_to(b2[None, :], (8, D_H))
    bcr = jnp.broadcast_to(bc[None, :], (8, 1))

    p = _sc_scatter_kernel()(g1, ei)
    g2 = _tc2(p, g1, dis, b1r, W2)

    q = _sc_scatter_kernel()(g2, ei)
    y = _tc3(q, g2, dis, b2r, Wc, bcr)
    return y


# speedup vs baseline: 1.9816x; 1.0479x over previous
import functools

import jax
import jax.numpy as jnp
from jax import lax
from jax.experimental import pallas as pl
from jax.experimental.pallas import tpu as pltpu
from jax.experimental.pallas import tpu_sc as plsc

N = 10000
E = 320000
D_IN = 128
D_H = 64

NC = 2
NS = 16
L = 16
NW = NC * NS

EPW = E // NW
K = 128
T_CH = EPW // K
TAIL = EPW - T_CH * K
N_PAD = 10240
HR = N_PAD // L
RPT = N_PAD // NS
CK = 128


@functools.cache
def _sc_degree_kernel():
    mesh = plsc.VectorSubcoreMesh(core_axis_name="c", subcore_axis_name="s")
    return pl.kernel(
        _sc_degree_body,
        out_type=jax.ShapeDtypeStruct((NC, N_PAD), jnp.float32),
        mesh=mesh,
        scratch_types=[
            pltpu.VMEM((EPW,), jnp.int32),
            pltpu.VMEM((N_PAD,), jnp.float32),
            pltpu.VMEM((N_PAD // CK, CK), jnp.int32),
            pltpu.VMEM_SHARED((N_PAD,), jnp.float32),
        ],
        compiler_params=pltpu.CompilerParams(
            needs_layout_passes=False, use_tc_tiling_on_sc=False),
    )


def _sc_degree_body(ei_hbm, out_hbm, dst_v, hist_v, row_idx, acc):
    c = lax.axis_index("c")
    s = lax.axis_index("s")
    w = c * NS + s
    zeros16 = jnp.zeros((L,), jnp.float32)
    ones16 = jnp.full((L,), 1.0, jnp.float32)
    iota16 = lax.iota(jnp.int32, L)

    def zero_row(i, _):
        hist_v[pl.ds(i * L, L)] = zeros16
        return 0
    lax.fori_loop(0, HR, zero_row, 0)

    pltpu.sync_copy(hist_v.at[pl.ds(0, N_PAD // NS)],
                    acc.at[pl.ds(s * (N_PAD // NS), N_PAD // NS)])

    def ri(j, _):
        def rk(kk, _):
            row_idx[j, pl.ds(kk * L, L)] = j * CK + kk * L + iota16
            return 0
        return lax.fori_loop(0, CK // L, rk, 0)
    lax.fori_loop(0, N_PAD // CK, ri, 0)

    plsc.subcore_barrier()

    pltpu.sync_copy(ei_hbm.at[1, pl.ds(w * EPW, EPW)], dst_v)

    def body(i, _):
        for u in range(8):
            idx = dst_v[pl.ds((i * 8 + u) * L, L)]
            plsc.addupdate_scatter(hist_v, [idx], ones16)
        return 0
    lax.fori_loop(0, EPW // (8 * L), body, 0)
    for u in range(EPW // L - (EPW // (8 * L)) * 8):
        idx = dst_v[pl.ds(((EPW // (8 * L)) * 8 + u) * L, L)]
        plsc.addupdate_scatter(hist_v, [idx], ones16)

    def comb(j, _):
        pltpu.sync_copy(hist_v.at[pl.ds(j * CK, CK)],
                        acc.at[row_idx.at[j]], add=True)
        return 0
    lax.fori_loop(0, N_PAD // CK, comb, 0)

    plsc.subcore_barrier()
    pltpu.sync_copy(acc.at[pl.ds(s * (N_PAD // NS), N_PAD // NS)],
                    out_hbm.at[c, pl.ds(s * (N_PAD // NS), N_PAD // NS)])


@functools.cache
def _sc_scatter_kernel():
    mesh = plsc.VectorSubcoreMesh(core_axis_name="c", subcore_axis_name="s")
    return pl.kernel(
        _sc_scatter_body,
        out_type=jax.ShapeDtypeStruct((NC, N_PAD, D_H), jnp.float32),
        mesh=mesh,
        scratch_types=[
            pltpu.VMEM((EPW,), jnp.int32),
            pltpu.VMEM((EPW,), jnp.int32),
            pltpu.VMEM((K, D_H), jnp.float32),
            pltpu.VMEM((64, D_H), jnp.float32),
            pltpu.VMEM_SHARED((N_PAD, D_H), jnp.float32),
            pltpu.SemaphoreType.DMA,
        ],
        compiler_params=pltpu.CompilerParams(
            needs_layout_passes=False, use_tc_tiling_on_sc=False),
    )


def _sc_scatter_body(g_hbm, ei_hbm, out_hbm,
                     src_v, dst_v, rows_v, zbuf, acc, sem):
    c = lax.axis_index("c")
    s = lax.axis_index("s")
    w = c * NS + s
    zeros16 = jnp.zeros((L,), jnp.float32)

    def zrow(i, _):
        def zcol(kk, _):
            zbuf[i, pl.ds(kk * L, L)] = zeros16
            return 0
        return lax.fori_loop(0, D_H // L, zcol, 0)
    lax.fori_loop(0, 64, zrow, 0)

    def zc(i, _):
        pltpu.sync_copy(zbuf, acc.at[pl.ds(s * RPT + i * 64, 64)])
        return 0
    lax.fori_loop(0, RPT // 64, zc, 0)

    plsc.subcore_barrier()

    pltpu.sync_copy(ei_hbm.at[0, pl.ds(w * EPW, EPW)], src_v)
    pltpu.sync_copy(ei_hbm.at[1, pl.ds(w * EPW, EPW)], dst_v)

    def body(j, _):
        pltpu.async_copy(g_hbm.at[src_v.at[pl.ds(j * K, K)]],
                         rows_v, sem).wait()
        pltpu.sync_copy(rows_v, acc.at[dst_v.at[pl.ds(j * K, K)]], add=True)
        return 0
    lax.fori_loop(0, T_CH, body, 0)

    pltpu.async_copy(g_hbm.at[src_v.at[pl.ds(T_CH * K, TAIL)]],
                     rows_v.at[pl.ds(0, TAIL)], sem).wait()
    pltpu.sync_copy(rows_v.at[pl.ds(0, TAIL)],
                    acc.at[dst_v.at[pl.ds(T_CH * K, TAIL)]], add=True)

    plsc.subcore_barrier()
    pltpu.sync_copy(acc.at[pl.ds(s * RPT, RPT)],
                    out_hbm.at[c, pl.ds(s * RPT, RPT)])


_R = 2048
_G = -(-N // _R)


def _dis_col(d_ref):
    deg = d_ref[0:1, :] + d_ref[1:2, :] + 1.0
    dis = lax.rsqrt(deg)
    return jnp.reshape(dis, (_R, 1))


def _tc1_body(x_ref, w_ref, d_ref, g_ref):
    h = jnp.dot(x_ref[...], w_ref[...], preferred_element_type=jnp.float32)
    g_ref[...] = h * _dis_col(d_ref)


def _tc1(x, W1, deg):
    return pl.pallas_call(
        _tc1_body,
        grid=(_G,),
        in_specs=[
            pl.BlockSpec((_R, D_IN), lambda m: (m, 0)),
            pl.BlockSpec((D_IN, D_H), lambda m: (0, 0)),
            pl.BlockSpec((NC, _R), lambda m: (0, m)),
        ],
        out_specs=pl.BlockSpec((_R, D_H), lambda m: (m, 0)),
        out_shape=jax.ShapeDtypeStruct((N, D_H), jnp.float32),
    )(x, W1, deg)


def _tc2_body(p_ref, g1_ref, d_ref, b1_ref, w2_ref, g2_ref):
    dis = _dis_col(d_ref)
    agg = p_ref[0] + p_ref[1] + g1_ref[...]
    h = jnp.maximum(agg * dis + b1_ref[0:1, :], 0.0)
    g2_ref[...] = jnp.dot(h, w2_ref[...],
                          preferred_element_type=jnp.float32) * dis


def _tc2(p, g1, deg, b1r, W2):
    return pl.pallas_call(
        _tc2_body,
        grid=(_G,),
        in_specs=[
            pl.BlockSpec((NC, _R, D_H), lambda m: (0, m, 0)),
            pl.BlockSpec((_R, D_H), lambda m: (m, 0)),
            pl.BlockSpec((NC, _R), lambda m: (0, m)),
            pl.BlockSpec((8, D_H), lambda m: (0, 0)),
            pl.BlockSpec((D_H, D_H), lambda m: (0, 0)),
        ],
        out_specs=pl.BlockSpec((_R, D_H), lambda m: (m, 0)),
        out_shape=jax.ShapeDtypeStruct((N, D_H), jnp.float32),
    )(p, g1, deg, b1r, W2)


def _tc3_body(p_ref, g2_ref, d_ref, b2_ref, wc_ref, bc_ref, y_ref):
    dis = _dis_col(d_ref)
    agg = p_ref[0] + p_ref[1] + g2_ref[...]
    h = jnp.maximum(agg * dis + b2_ref[0:1, :], 0.0)
    z = jnp.dot(h, wc_ref[...],
                preferred_element_type=jnp.float32) + bc_ref[0:1, 0:1]
    y_ref[...] = jax.nn.sigmoid(z)


def _tc3(p, g2, deg, b2r, Wc, bcr):
    return pl.pallas_call(
        _tc3_body,
        grid=(_G,),
        in_specs=[
            pl.BlockSpec((NC, _R, D_H), lambda m: (0, m, 0)),
            pl.BlockSpec((_R, D_H), lambda m: (m, 0)),
            pl.BlockSpec((NC, _R), lambda m: (0, m)),
            pl.BlockSpec((8, D_H), lambda m: (0, 0)),
            pl.BlockSpec((D_H, 1), lambda m: (0, 0)),
            pl.BlockSpec((8, 1), lambda m: (0, 0)),
        ],
        out_specs=pl.BlockSpec((_R, 1), lambda m: (m, 0)),
        out_shape=jax.ShapeDtypeStruct((N, 1), jnp.float32),
    )(p, g2, deg, b2r, Wc, bcr)


def kernel(x, edge_index, W1, b1, W2, b2, Wc, bc):
    ei = edge_index.astype(jnp.int32)

    deg = _sc_degree_kernel()(ei)

    g1 = _tc1(x, W1, deg)

    b1r = jnp.broadcast_to(b1[None, :], (8, D_H))
    b2r = jnp.broadcast_to(b2[None, :], (8, D_H))
    bcr = jnp.broadcast_to(bc[None, :], (8, 1))

    p = _sc_scatter_kernel()(g1, ei)
    g2 = _tc2(p, g1, deg, b1r, W2)

    q = _sc_scatter_kernel()(g2, ei)
    y = _tc3(q, g2, deg, b2r, Wc, bcr)
    return y


# speedup vs baseline: 2.0279x; 1.0234x over previous
import functools

import jax
import jax.numpy as jnp
from jax import lax
from jax.experimental import pallas as pl
from jax.experimental.pallas import tpu as pltpu
from jax.experimental.pallas import tpu_sc as plsc

N = 10000
E = 320000
D_IN = 128
D_H = 64

NC = 2
NS = 16
L = 16
NW = NC * NS

EPW = E // NW
K = 128
T_CH = EPW // K
TAIL = EPW - T_CH * K
N_PAD = 10240
HR = N_PAD // L
RPT = N_PAD // NS
CK = 128


@functools.cache
def _sc_degree_kernel():
    mesh = plsc.VectorSubcoreMesh(core_axis_name="c", subcore_axis_name="s")
    return pl.kernel(
        _sc_degree_body,
        out_type=jax.ShapeDtypeStruct((NC, N_PAD), jnp.float32),
        mesh=mesh,
        scratch_types=[
            pltpu.VMEM((EPW,), jnp.int32),
            pltpu.VMEM((N_PAD,), jnp.float32),
            pltpu.VMEM((N_PAD // CK, CK), jnp.int32),
            pltpu.VMEM_SHARED((N_PAD,), jnp.float32),
        ],
        compiler_params=pltpu.CompilerParams(
            needs_layout_passes=False, use_tc_tiling_on_sc=False),
    )


def _sc_degree_body(ei_hbm, out_hbm, dst_v, hist_v, row_idx, acc):
    c = lax.axis_index("c")
    s = lax.axis_index("s")
    w = c * NS + s
    zeros16 = jnp.zeros((L,), jnp.float32)
    ones16 = jnp.full((L,), 1.0, jnp.float32)
    iota16 = lax.iota(jnp.int32, L)

    def zero_row(i, _):
        hist_v[pl.ds(i * L, L)] = zeros16
        return 0
    lax.fori_loop(0, HR, zero_row, 0)

    pltpu.sync_copy(hist_v.at[pl.ds(0, N_PAD // NS)],
                    acc.at[pl.ds(s * (N_PAD // NS), N_PAD // NS)])

    def ri(j, _):
        def rk(kk, _):
            row_idx[j, pl.ds(kk * L, L)] = j * CK + kk * L + iota16
            return 0
        return lax.fori_loop(0, CK // L, rk, 0)
    lax.fori_loop(0, N_PAD // CK, ri, 0)

    plsc.subcore_barrier()

    pltpu.sync_copy(ei_hbm.at[1, pl.ds(w * EPW, EPW)], dst_v)

    def body(i, _):
        for u in range(8):
            idx = dst_v[pl.ds((i * 8 + u) * L, L)]
            plsc.addupdate_scatter(hist_v, [idx], ones16)
        return 0
    lax.fori_loop(0, EPW // (8 * L), body, 0)
    for u in range(EPW // L - (EPW // (8 * L)) * 8):
        idx = dst_v[pl.ds(((EPW // (8 * L)) * 8 + u) * L, L)]
        plsc.addupdate_scatter(hist_v, [idx], ones16)

    def comb(j, _):
        pltpu.sync_copy(hist_v.at[pl.ds(j * CK, CK)],
                        acc.at[row_idx.at[j]], add=True)
        return 0
    lax.fori_loop(0, N_PAD // CK, comb, 0)

    plsc.subcore_barrier()
    pltpu.sync_copy(acc.at[pl.ds(s * (N_PAD // NS), N_PAD // NS)],
                    out_hbm.at[c, pl.ds(s * (N_PAD // NS), N_PAD // NS)])


@functools.cache
def _sc_scatter_kernel():
    mesh = plsc.VectorSubcoreMesh(core_axis_name="c", subcore_axis_name="s")
    return pl.kernel(
        _sc_scatter_body,
        out_type=jax.ShapeDtypeStruct((NC, N_PAD, D_H), jnp.float32),
        mesh=mesh,
        scratch_types=[
            pltpu.VMEM((EPW,), jnp.int32),
            pltpu.VMEM((EPW,), jnp.int32),
            pltpu.VMEM((K, D_H), jnp.float32),
            pltpu.VMEM((64, D_H), jnp.float32),
            pltpu.VMEM_SHARED((N_PAD, D_H), jnp.float32),
            pltpu.VMEM_SHARED((N, D_H), jnp.float32),
            pltpu.SemaphoreType.DMA,
        ],
        compiler_params=pltpu.CompilerParams(
            needs_layout_passes=False, use_tc_tiling_on_sc=False),
    )


def _sc_scatter_body(g_hbm, ei_hbm, out_hbm,
                     src_v, dst_v, rows_v, zbuf, acc, g_sh, sem):
    c = lax.axis_index("c")
    s = lax.axis_index("s")
    w = c * NS + s
    zeros16 = jnp.zeros((L,), jnp.float32)

    def zrow(i, _):
        def zcol(kk, _):
            zbuf[i, pl.ds(kk * L, L)] = zeros16
            return 0
        return lax.fori_loop(0, D_H // L, zcol, 0)
    lax.fori_loop(0, 64, zrow, 0)

    def zc(i, _):
        pltpu.sync_copy(zbuf, acc.at[pl.ds(s * RPT + i * 64, 64)])
        return 0
    lax.fori_loop(0, RPT // 64, zc, 0)

    pltpu.sync_copy(g_hbm.at[pl.ds(s * (N // NS), N // NS)],
                    g_sh.at[pl.ds(s * (N // NS), N // NS)])

    plsc.subcore_barrier()

    pltpu.sync_copy(ei_hbm.at[0, pl.ds(w * EPW, EPW)], src_v)
    pltpu.sync_copy(ei_hbm.at[1, pl.ds(w * EPW, EPW)], dst_v)

    def body(j, _):
        pltpu.async_copy(g_sh.at[src_v.at[pl.ds(j * K, K)]],
                         rows_v, sem).wait()
        pltpu.sync_copy(rows_v, acc.at[dst_v.at[pl.ds(j * K, K)]], add=True)
        return 0
    lax.fori_loop(0, T_CH, body, 0)

    pltpu.async_copy(g_sh.at[src_v.at[pl.ds(T_CH * K, TAIL)]],
                     rows_v.at[pl.ds(0, TAIL)], sem).wait()
    pltpu.sync_copy(rows_v.at[pl.ds(0, TAIL)],
                    acc.at[dst_v.at[pl.ds(T_CH * K, TAIL)]], add=True)

    plsc.subcore_barrier()
    pltpu.sync_copy(acc.at[pl.ds(s * RPT, RPT)],
                    out_hbm.at[c, pl.ds(s * RPT, RPT)])


_R = 2048
_G = -(-N // _R)


def _dis_col(d_ref):
    deg = d_ref[0:1, :] + d_ref[1:2, :] + 1.0
    dis = lax.rsqrt(deg)
    return jnp.reshape(dis, (_R, 1))


def _tc1_body(x_ref, w_ref, d_ref, g_ref):
    h = jnp.dot(x_ref[...], w_ref[...], preferred_element_type=jnp.float32)
    g_ref[...] = h * _dis_col(d_ref)


def _tc1(x, W1, deg):
    return pl.pallas_call(
        _tc1_body,
        grid=(_G,),
        in_specs=[
            pl.BlockSpec((_R, D_IN), lambda m: (m, 0)),
            pl.BlockSpec((D_IN, D_H), lambda m: (0, 0)),
            pl.BlockSpec((NC, _R), lambda m: (0, m)),
        ],
        out_specs=pl.BlockSpec((_R, D_H), lambda m: (m, 0)),
        out_shape=jax.ShapeDtypeStruct((N, D_H), jnp.float32),
    )(x, W1, deg)


def _tc2_body(p_ref, g1_ref, d_ref, b1_ref, w2_ref, g2_ref):
    dis = _dis_col(d_ref)
    agg = p_ref[0] + p_ref[1] + g1_ref[...]
    h = jnp.maximum(agg * dis + b1_ref[0:1, :], 0.0)
    g2_ref[...] = jnp.dot(h, w2_ref[...],
                          preferred_element_type=jnp.float32) * dis


def _tc2(p, g1, deg, b1r, W2):
    return pl.pallas_call(
        _tc2_body,
        grid=(_G,),
        in_specs=[
            pl.BlockSpec((NC, _R, D_H), lambda m: (0, m, 0)),
            pl.BlockSpec((_R, D_H), lambda m: (m, 0)),
            pl.BlockSpec((NC, _R), lambda m: (0, m)),
            pl.BlockSpec((8, D_H), lambda m: (0, 0)),
            pl.BlockSpec((D_H, D_H), lambda m: (0, 0)),
        ],
        out_specs=pl.BlockSpec((_R, D_H), lambda m: (m, 0)),
        out_shape=jax.ShapeDtypeStruct((N, D_H), jnp.float32),
    )(p, g1, deg, b1r, W2)


def _tc3_body(p_ref, g2_ref, d_ref, b2_ref, wc_ref, bc_ref, y_ref):
    dis = _dis_col(d_ref)
    agg = p_ref[0] + p_ref[1] + g2_ref[...]
    h = jnp.maximum(agg * dis + b2_ref[0:1, :], 0.0)
    z = jnp.dot(h, wc_ref[...],
                preferred_element_type=jnp.float32) + bc_ref[0:1, 0:1]
    y_ref[...] = jax.nn.sigmoid(z)


def _tc3(p, g2, deg, b2r, Wc, bcr):
    return pl.pallas_call(
        _tc3_body,
        grid=(_G,),
        in_specs=[
            pl.BlockSpec((NC, _R, D_H), lambda m: (0, m, 0)),
            pl.BlockSpec((_R, D_H), lambda m: (m, 0)),
            pl.BlockSpec((NC, _R), lambda m: (0, m)),
            pl.BlockSpec((8, D_H), lambda m: (0, 0)),
            pl.BlockSpec((D_H, 1), lambda m: (0, 0)),
            pl.BlockSpec((8, 1), lambda m: (0, 0)),
        ],
        out_specs=pl.BlockSpec((_R, 1), lambda m: (m, 0)),
        out_shape=jax.ShapeDtypeStruct((N, 1), jnp.float32),
    )(p, g2, deg, b2r, Wc, bcr)


def kernel(x, edge_index, W1, b1, W2, b2, Wc, bc):
    ei = edge_index.astype(jnp.int32)

    deg = _sc_degree_kernel()(ei)

    g1 = _tc1(x, W1, deg)

    b1r = jnp.broadcast_to(b1[None, :], (8, D_H))
    b2r = jnp.broadcast_to(b2[None, :], (8, D_H))
    bcr = jnp.broadcast_to(bc[None, :], (8, 1))

    p = _sc_scatter_kernel()(g1, ei)
    g2 = _tc2(p, g1, deg, b1r, W2)

    q = _sc_scatter_kernel()(g2, ei)
    y = _tc3(q, g2, deg, b2r, Wc, bcr)
    return y


# speedup vs baseline: 2.3661x; 1.1668x over previous
import functools

import jax
import jax.numpy as jnp
from jax import lax
from jax.experimental import pallas as pl
from jax.experimental.pallas import tpu as pltpu
from jax.experimental.pallas import tpu_sc as plsc

N = 10000
E = 320000
D_IN = 128
D_H = 64

NC = 2
NS = 16
L = 16
NW = NC * NS

EPW = E // NW
K = 128
T_CH = EPW // K
TAIL = EPW - T_CH * K
N_PAD = 10240
HR = N_PAD // L
RPT = N_PAD // NS
CK = 128


@functools.cache
def _sc_degree_kernel():
    mesh = plsc.VectorSubcoreMesh(core_axis_name="c", subcore_axis_name="s")
    return pl.kernel(
        _sc_degree_body,
        out_type=jax.ShapeDtypeStruct((NC, N_PAD), jnp.float32),
        mesh=mesh,
        scratch_types=[
            pltpu.VMEM((EPW,), jnp.int32),
            pltpu.VMEM((N_PAD,), jnp.float32),
            pltpu.VMEM((N_PAD // CK, CK), jnp.int32),
            pltpu.VMEM_SHARED((N_PAD,), jnp.float32),
        ],
        compiler_params=pltpu.CompilerParams(
            needs_layout_passes=False, use_tc_tiling_on_sc=False),
    )


def _sc_degree_body(ei_hbm, out_hbm, dst_v, hist_v, row_idx, acc):
    c = lax.axis_index("c")
    s = lax.axis_index("s")
    w = c * NS + s
    zeros16 = jnp.zeros((L,), jnp.float32)
    ones16 = jnp.full((L,), 1.0, jnp.float32)
    iota16 = lax.iota(jnp.int32, L)

    def zero_row(i, _):
        hist_v[pl.ds(i * L, L)] = zeros16
        return 0
    lax.fori_loop(0, HR, zero_row, 0)

    pltpu.sync_copy(hist_v.at[pl.ds(0, N_PAD // NS)],
                    acc.at[pl.ds(s * (N_PAD // NS), N_PAD // NS)])

    def ri(j, _):
        def rk(kk, _):
            row_idx[j, pl.ds(kk * L, L)] = j * CK + kk * L + iota16
            return 0
        return lax.fori_loop(0, CK // L, rk, 0)
    lax.fori_loop(0, N_PAD // CK, ri, 0)

    plsc.subcore_barrier()

    pltpu.sync_copy(ei_hbm.at[1, pl.ds(w * EPW, EPW)], dst_v)

    def body(i, _):
        for u in range(8):
            idx = dst_v[pl.ds((i * 8 + u) * L, L)]
            plsc.addupdate_scatter(hist_v, [idx], ones16)
        return 0
    lax.fori_loop(0, EPW // (8 * L), body, 0)
    for u in range(EPW // L - (EPW // (8 * L)) * 8):
        idx = dst_v[pl.ds(((EPW // (8 * L)) * 8 + u) * L, L)]
        plsc.addupdate_scatter(hist_v, [idx], ones16)

    def comb(j, _):
        pltpu.sync_copy(hist_v.at[pl.ds(j * CK, CK)],
                        acc.at[row_idx.at[j]], add=True)
        return 0
    lax.fori_loop(0, N_PAD // CK, comb, 0)

    plsc.subcore_barrier()
    pltpu.sync_copy(acc.at[pl.ds(s * (N_PAD // NS), N_PAD // NS)],
                    out_hbm.at[c, pl.ds(s * (N_PAD // NS), N_PAD // NS)])


@functools.cache
def _sc_scatter_kernel():
    mesh = plsc.VectorSubcoreMesh(core_axis_name="c", subcore_axis_name="s")
    return pl.kernel(
        _sc_scatter_body,
        out_type=jax.ShapeDtypeStruct((NC, N_PAD, D_H), jnp.float32),
        mesh=mesh,
        scratch_types=[
            pltpu.VMEM((EPW,), jnp.int32),
            pltpu.VMEM((EPW,), jnp.int32),
            pltpu.VMEM((2, K, D_H), jnp.float32),
            pltpu.VMEM((64, D_H), jnp.float32),
            pltpu.VMEM_SHARED((N_PAD, D_H), jnp.float32),
            pltpu.SemaphoreType.DMA,
            pltpu.SemaphoreType.DMA,
        ],
        compiler_params=pltpu.CompilerParams(
            needs_layout_passes=False, use_tc_tiling_on_sc=False),
    )


def _sc_scatter_body(g_hbm, ei_hbm, out_hbm,
                     src_v, dst_v, rows2, zbuf, acc, sem_g, sem_s):
    c = lax.axis_index("c")
    s = lax.axis_index("s")
    w = c * NS + s
    zeros16 = jnp.zeros((L,), jnp.float32)

    def zrow(i, _):
        def zcol(kk, _):
            zbuf[i, pl.ds(kk * L, L)] = zeros16
            return 0
        return lax.fori_loop(0, D_H // L, zcol, 0)
    lax.fori_loop(0, 64, zrow, 0)

    def zc(i, _):
        pltpu.sync_copy(zbuf, acc.at[pl.ds(s * RPT + i * 64, 64)])
        return 0
    lax.fori_loop(0, RPT // 64, zc, 0)

    plsc.subcore_barrier()

    pltpu.sync_copy(ei_hbm.at[0, pl.ds(w * EPW, EPW)], src_v)
    pltpu.sync_copy(ei_hbm.at[1, pl.ds(w * EPW, EPW)], dst_v)

    def gather(j, b):
        pltpu.async_copy(g_hbm.at[src_v.at[pl.ds(j * K, K)]],
                         rows2.at[b], sem_g).wait()

    def scat_start(j, b):
        pltpu.async_copy(rows2.at[b], acc.at[dst_v.at[pl.ds(j * K, K)]],
                         sem_s, add=True)

    def scat_wait(j, b):
        pltpu.make_async_copy(rows2.at[b],
                              acc.at[dst_v.at[pl.ds(j * K, K)]],
                              sem_s).wait()

    gather(0, 0)

    def body(j, _):
        b = lax.bitwise_and(j, 1)
        scat_start(j, b)
        gather(j + 1, 1 - b)
        scat_wait(j, b)
        return 0
    lax.fori_loop(0, T_CH - 1, body, 0)

    bl = (T_CH - 1) % 2
    scat_start(T_CH - 1, bl)
    pltpu.async_copy(g_hbm.at[src_v.at[pl.ds(T_CH * K, TAIL)]],
                     rows2.at[1 - bl].at[pl.ds(0, TAIL)], sem_g).wait()
    scat_wait(T_CH - 1, bl)
    pltpu.sync_copy(rows2.at[1 - bl].at[pl.ds(0, TAIL)],
                    acc.at[dst_v.at[pl.ds(T_CH * K, TAIL)]], add=True)

    plsc.subcore_barrier()
    pltpu.sync_copy(acc.at[pl.ds(s * RPT, RPT)],
                    out_hbm.at[c, pl.ds(s * RPT, RPT)])


_R = 2048
_G = -(-N // _R)


def _dis_col(d_ref):
    deg = d_ref[0:1, :] + d_ref[1:2, :] + 1.0
    dis = lax.rsqrt(deg)
    return jnp.reshape(dis, (_R, 1))


def _tc1_body(x_ref, w_ref, d_ref, g_ref):
    h = jnp.dot(x_ref[...], w_ref[...], preferred_element_type=jnp.float32)
    g_ref[...] = h * _dis_col(d_ref)


def _tc1(x, W1, deg):
    return pl.pallas_call(
        _tc1_body,
        grid=(_G,),
        in_specs=[
            pl.BlockSpec((_R, D_IN), lambda m: (m, 0)),
            pl.BlockSpec((D_IN, D_H), lambda m: (0, 0)),
            pl.BlockSpec((NC, _R), lambda m: (0, m)),
        ],
        out_specs=pl.BlockSpec((_R, D_H), lambda m: (m, 0)),
        out_shape=jax.ShapeDtypeStruct((N, D_H), jnp.float32),
    )(x, W1, deg)


def _tc2_body(p_ref, g1_ref, d_ref, b1_ref, w2_ref, g2_ref):
    dis = _dis_col(d_ref)
    agg = p_ref[0] + p_ref[1] + g1_ref[...]
    h = jnp.maximum(agg * dis + b1_ref[0:1, :], 0.0)
    g2_ref[...] = jnp.dot(h, w2_ref[...],
                          preferred_element_type=jnp.float32) * dis


def _tc2(p, g1, deg, b1r, W2):
    return pl.pallas_call(
        _tc2_body,
        grid=(_G,),
        in_specs=[
            pl.BlockSpec((NC, _R, D_H), lambda m: (0, m, 0)),
            pl.BlockSpec((_R, D_H), lambda m: (m, 0)),
            pl.BlockSpec((NC, _R), lambda m: (0, m)),
            pl.BlockSpec((8, D_H), lambda m: (0, 0)),
            pl.BlockSpec((D_H, D_H), lambda m: (0, 0)),
        ],
        out_specs=pl.BlockSpec((_R, D_H), lambda m: (m, 0)),
        out_shape=jax.ShapeDtypeStruct((N, D_H), jnp.float32),
    )(p, g1, deg, b1r, W2)


def _tc3_body(p_ref, g2_ref, d_ref, b2_ref, wc_ref, bc_ref, y_ref):
    dis = _dis_col(d_ref)
    agg = p_ref[0] + p_ref[1] + g2_ref[...]
    h = jnp.maximum(agg * dis + b2_ref[0:1, :], 0.0)
    z = jnp.dot(h, wc_ref[...],
                preferred_element_type=jnp.float32) + bc_ref[0:1, 0:1]
    y_ref[...] = jax.nn.sigmoid(z)


def _tc3(p, g2, deg, b2r, Wc, bcr):
    return pl.pallas_call(
        _tc3_body,
        grid=(_G,),
        in_specs=[
            pl.BlockSpec((NC, _R, D_H), lambda m: (0, m, 0)),
            pl.BlockSpec((_R, D_H), lambda m: (m, 0)),
            pl.BlockSpec((NC, _R), lambda m: (0, m)),
            pl.BlockSpec((8, D_H), lambda m: (0, 0)),
            pl.BlockSpec((D_H, 1), lambda m: (0, 0)),
            pl.BlockSpec((8, 1), lambda m: (0, 0)),
        ],
        out_specs=pl.BlockSpec((_R, 1), lambda m: (m, 0)),
        out_shape=jax.ShapeDtypeStruct((N, 1), jnp.float32),
    )(p, g2, deg, b2r, Wc, bcr)


def kernel(x, edge_index, W1, b1, W2, b2, Wc, bc):
    ei = edge_index.astype(jnp.int32)

    deg = _sc_degree_kernel()(ei)

    g1 = _tc1(x, W1, deg)

    b1r = jnp.broadcast_to(b1[None, :], (8, D_H))
    b2r = jnp.broadcast_to(b2[None, :], (8, D_H))
    bcr = jnp.broadcast_to(bc[None, :], (8, 1))

    p = _sc_scatter_kernel()(g1, ei)
    g2 = _tc2(p, g1, deg, b1r, W2)

    q = _sc_scatter_kernel()(g2, ei)
    y = _tc3(q, g2, deg, b2r, Wc, bcr)
    return y


# speedup vs baseline: 2.3684x; 1.0009x over previous
import functools

import jax
import jax.numpy as jnp
from jax import lax
from jax.experimental import pallas as pl
from jax.experimental.pallas import tpu as pltpu
from jax.experimental.pallas import tpu_sc as plsc

N = 10000
E = 320000
D_IN = 128
D_H = 64

NC = 2
NS = 16
L = 16
NW = NC * NS

EPW = E // NW
K = 128
T_CH = EPW // K
TAIL = EPW - T_CH * K
N_PAD = 10240
HR = N_PAD // L
RPT = N_PAD // NS
CK = 128


@functools.cache
def _sc_degree_kernel():
    mesh = plsc.VectorSubcoreMesh(core_axis_name="c", subcore_axis_name="s")
    return pl.kernel(
        _sc_degree_body,
        out_type=jax.ShapeDtypeStruct((NC, N_PAD), jnp.float32),
        mesh=mesh,
        scratch_types=[
            pltpu.VMEM((EPW,), jnp.int32),
            pltpu.VMEM((N_PAD,), jnp.float32),
            pltpu.VMEM((N_PAD // CK, CK), jnp.int32),
            pltpu.VMEM_SHARED((N_PAD,), jnp.float32),
        ],
        compiler_params=pltpu.CompilerParams(
            needs_layout_passes=False, use_tc_tiling_on_sc=False),
    )


def _sc_degree_body(ei_hbm, out_hbm, dst_v, hist_v, row_idx, acc):
    c = lax.axis_index("c")
    s = lax.axis_index("s")
    w = c * NS + s
    zeros16 = jnp.zeros((L,), jnp.float32)
    ones16 = jnp.full((L,), 1.0, jnp.float32)
    iota16 = lax.iota(jnp.int32, L)

    def zero_row(i, _):
        hist_v[pl.ds(i * L, L)] = zeros16
        return 0
    lax.fori_loop(0, HR, zero_row, 0)

    pltpu.sync_copy(hist_v.at[pl.ds(0, N_PAD // NS)],
                    acc.at[pl.ds(s * (N_PAD // NS), N_PAD // NS)])

    def ri(j, _):
        def rk(kk, _):
            row_idx[j, pl.ds(kk * L, L)] = j * CK + kk * L + iota16
            return 0
        return lax.fori_loop(0, CK // L, rk, 0)
    lax.fori_loop(0, N_PAD // CK, ri, 0)

    plsc.subcore_barrier()

    pltpu.sync_copy(ei_hbm.at[1, pl.ds(w * EPW, EPW)], dst_v)

    def body(i, _):
        for u in range(8):
            idx = dst_v[pl.ds((i * 8 + u) * L, L)]
            plsc.addupdate_scatter(hist_v, [idx], ones16)
        return 0
    lax.fori_loop(0, EPW // (8 * L), body, 0)
    for u in range(EPW // L - (EPW // (8 * L)) * 8):
        idx = dst_v[pl.ds(((EPW // (8 * L)) * 8 + u) * L, L)]
        plsc.addupdate_scatter(hist_v, [idx], ones16)

    def comb(j, _):
        pltpu.sync_copy(hist_v.at[pl.ds(j * CK, CK)],
                        acc.at[row_idx.at[j]], add=True)
        return 0
    lax.fori_loop(0, N_PAD // CK, comb, 0)

    plsc.subcore_barrier()
    pltpu.sync_copy(acc.at[pl.ds(s * (N_PAD // NS), N_PAD // NS)],
                    out_hbm.at[c, pl.ds(s * (N_PAD // NS), N_PAD // NS)])


@functools.cache
def _sc_scatter_kernel():
    mesh = plsc.VectorSubcoreMesh(core_axis_name="c", subcore_axis_name="s")
    return pl.kernel(
        _sc_scatter_body,
        out_type=jax.ShapeDtypeStruct((NC, N_PAD, D_H), jnp.float32),
        mesh=mesh,
        scratch_types=[
            pltpu.VMEM((EPW,), jnp.int32),
            pltpu.VMEM((EPW,), jnp.int32),
            pltpu.VMEM((2, K, D_H), jnp.float32),
            pltpu.VMEM((64, D_H), jnp.float32),
            pltpu.VMEM_SHARED((N_PAD, D_H), jnp.float32),
            pltpu.SemaphoreType.DMA,
            pltpu.SemaphoreType.DMA,
        ],
        compiler_params=pltpu.CompilerParams(
            needs_layout_passes=False, use_tc_tiling_on_sc=False),
    )


def _sc_scatter_body(g_hbm, ei_hbm, out_hbm,
                     src_v, dst_v, rows2, zbuf, acc, sem_g, sem_s):
    c = lax.axis_index("c")
    s = lax.axis_index("s")
    w = c * NS + s
    zeros16 = jnp.zeros((L,), jnp.float32)

    def zrow(i, _):
        def zcol(kk, _):
            zbuf[i, pl.ds(kk * L, L)] = zeros16
            return 0
        return lax.fori_loop(0, D_H // L, zcol, 0)
    lax.fori_loop(0, 64, zrow, 0)

    def zc(i, _):
        pltpu.sync_copy(zbuf, acc.at[pl.ds(s * RPT + i * 64, 64)])
        return 0
    lax.fori_loop(0, RPT // 64, zc, 0)

    plsc.subcore_barrier()

    pltpu.sync_copy(ei_hbm.at[0, pl.ds(w * EPW, EPW)], src_v)
    pltpu.sync_copy(ei_hbm.at[1, pl.ds(w * EPW, EPW)], dst_v)

    def gather(j, b):
        pltpu.async_copy(g_hbm.at[src_v.at[pl.ds(j * K, K)]],
                         rows2.at[b], sem_g).wait()

    def scat_start(j, b):
        pltpu.async_copy(rows2.at[b], acc.at[dst_v.at[pl.ds(j * K, K)]],
                         sem_s, add=True)

    def scat_wait(j, b):
        pltpu.make_async_copy(rows2.at[b],
                              acc.at[dst_v.at[pl.ds(j * K, K)]],
                              sem_s).wait()

    gather(0, 0)

    def body(j, _):
        b = lax.bitwise_and(j, 1)
        scat_start(j, b)
        gather(j + 1, 1 - b)
        scat_wait(j, b)
        return 0
    lax.fori_loop(0, T_CH - 1, body, 0)

    bl = (T_CH - 1) % 2
    scat_start(T_CH - 1, bl)
    pltpu.async_copy(g_hbm.at[src_v.at[pl.ds(T_CH * K, TAIL)]],
                     rows2.at[1 - bl].at[pl.ds(0, TAIL)], sem_g).wait()
    scat_wait(T_CH - 1, bl)
    pltpu.sync_copy(rows2.at[1 - bl].at[pl.ds(0, TAIL)],
                    acc.at[dst_v.at[pl.ds(T_CH * K, TAIL)]], add=True)

    plsc.subcore_barrier()
    pltpu.sync_copy(acc.at[pl.ds(s * RPT, RPT)],
                    out_hbm.at[c, pl.ds(s * RPT, RPT)])


_R = 2048
_G = -(-N // _R)


def _dis_col(d_ref):
    deg = d_ref[0:1, :] + d_ref[1:2, :] + 1.0
    dis = lax.rsqrt(deg)
    return jnp.reshape(dis, (_R, 1))


def _tc1a_body(x_ref, w_ref, h_ref):
    h_ref[...] = jnp.dot(x_ref[...], w_ref[...],
                         preferred_element_type=jnp.float32)


def _tc1a(x, W1):
    return pl.pallas_call(
        _tc1a_body,
        grid=(_G,),
        in_specs=[
            pl.BlockSpec((_R, D_IN), lambda m: (m, 0)),
            pl.BlockSpec((D_IN, D_H), lambda m: (0, 0)),
        ],
        out_specs=pl.BlockSpec((_R, D_H), lambda m: (m, 0)),
        out_shape=jax.ShapeDtypeStruct((N, D_H), jnp.float32),
    )(x, W1)


def _tc1b_body(h_ref, d_ref, g_ref):
    g_ref[...] = h_ref[...] * _dis_col(d_ref)


def _tc1b(h, deg):
    return pl.pallas_call(
        _tc1b_body,
        grid=(_G,),
        in_specs=[
            pl.BlockSpec((_R, D_H), lambda m: (m, 0)),
            pl.BlockSpec((NC, _R), lambda m: (0, m)),
        ],
        out_specs=pl.BlockSpec((_R, D_H), lambda m: (m, 0)),
        out_shape=jax.ShapeDtypeStruct((N, D_H), jnp.float32),
    )(h, deg)


def _tc2_body(p_ref, g1_ref, d_ref, b1_ref, w2_ref, g2_ref):
    dis = _dis_col(d_ref)
    agg = p_ref[0] + p_ref[1] + g1_ref[...]
    h = jnp.maximum(agg * dis + b1_ref[0:1, :], 0.0)
    g2_ref[...] = jnp.dot(h, w2_ref[...],
                          preferred_element_type=jnp.float32) * dis


def _tc2(p, g1, deg, b1r, W2):
    return pl.pallas_call(
        _tc2_body,
        grid=(_G,),
        in_specs=[
            pl.BlockSpec((NC, _R, D_H), lambda m: (0, m, 0)),
            pl.BlockSpec((_R, D_H), lambda m: (m, 0)),
            pl.BlockSpec((NC, _R), lambda m: (0, m)),
            pl.BlockSpec((8, D_H), lambda m: (0, 0)),
            pl.BlockSpec((D_H, D_H), lambda m: (0, 0)),
        ],
        out_specs=pl.BlockSpec((_R, D_H), lambda m: (m, 0)),
        out_shape=jax.ShapeDtypeStruct((N, D_H), jnp.float32),
    )(p, g1, deg, b1r, W2)


def _tc3_body(p_ref, g2_ref, d_ref, b2_ref, wc_ref, bc_ref, y_ref):
    dis = _dis_col(d_ref)
    agg = p_ref[0] + p_ref[1] + g2_ref[...]
    h = jnp.maximum(agg * dis + b2_ref[0:1, :], 0.0)
    z = jnp.dot(h, wc_ref[...],
                preferred_element_type=jnp.float32) + bc_ref[0:1, 0:1]
    y_ref[...] = jax.nn.sigmoid(z)


def _tc3(p, g2, deg, b2r, Wc, bcr):
    return pl.pallas_call(
        _tc3_body,
        grid=(_G,),
        in_specs=[
            pl.BlockSpec((NC, _R, D_H), lambda m: (0, m, 0)),
            pl.BlockSpec((_R, D_H), lambda m: (m, 0)),
            pl.BlockSpec((NC, _R), lambda m: (0, m)),
            pl.BlockSpec((8, D_H), lambda m: (0, 0)),
            pl.BlockSpec((D_H, 1), lambda m: (0, 0)),
            pl.BlockSpec((8, 1), lambda m: (0, 0)),
        ],
        out_specs=pl.BlockSpec((_R, 1), lambda m: (m, 0)),
        out_shape=jax.ShapeDtypeStruct((N, 1), jnp.float32),
    )(p, g2, deg, b2r, Wc, bcr)


def kernel(x, edge_index, W1, b1, W2, b2, Wc, bc):
    ei = edge_index.astype(jnp.int32)

    h1 = _tc1a(x, W1)
    deg = _sc_degree_kernel()(ei)
    g1 = _tc1b(h1, deg)

    b1r = jnp.broadcast_to(b1[None, :], (8, D_H))
    b2r = jnp.broadcast_to(b2[None, :], (8, D_H))
    bcr = jnp.broadcast_to(bc[None, :], (8, 1))

    p = _sc_scatter_kernel()(g1, ei)
    g2 = _tc2(p, g1, deg, b1r, W2)

    q = _sc_scatter_kernel()(g2, ei)
    y = _tc3(q, g2, deg, b2r, Wc, bcr)
    return y


# speedup vs baseline: 2.3699x; 1.0007x over previous
import functools

import jax
import jax.numpy as jnp
from jax import lax
from jax.experimental import pallas as pl
from jax.experimental.pallas import tpu as pltpu
from jax.experimental.pallas import tpu_sc as plsc

N = 10000
E = 320000
D_IN = 128
D_H = 64

NC = 2
NS = 16
L = 16
NW = NC * NS

EPW = E // NW
K = 128
T_CH = EPW // K
TAIL = EPW - T_CH * K
N_PAD = 10240
HR = N_PAD // L
RPT = N_PAD // NS
CK = 128


@functools.cache
def _sc_degree_kernel():
    mesh = plsc.VectorSubcoreMesh(core_axis_name="c", subcore_axis_name="s")
    return pl.kernel(
        _sc_degree_body,
        out_type=jax.ShapeDtypeStruct((NC, N_PAD), jnp.float32),
        mesh=mesh,
        scratch_types=[
            pltpu.VMEM((EPW,), jnp.int32),
            pltpu.VMEM((N_PAD,), jnp.float32),
            pltpu.VMEM((N_PAD // CK, CK), jnp.int32),
            pltpu.VMEM_SHARED((N_PAD,), jnp.float32),
        ],
        compiler_params=pltpu.CompilerParams(
            needs_layout_passes=False, use_tc_tiling_on_sc=False),
    )


def _sc_degree_body(ei_hbm, out_hbm, dst_v, hist_v, row_idx, acc):
    c = lax.axis_index("c")
    s = lax.axis_index("s")
    w = c * NS + s
    zeros16 = jnp.zeros((L,), jnp.float32)
    ones16 = jnp.full((L,), 1.0, jnp.float32)
    iota16 = lax.iota(jnp.int32, L)

    def zero_row(i, _):
        hist_v[pl.ds(i * L, L)] = zeros16
        return 0
    lax.fori_loop(0, HR, zero_row, 0)

    pltpu.sync_copy(hist_v.at[pl.ds(0, N_PAD // NS)],
                    acc.at[pl.ds(s * (N_PAD // NS), N_PAD // NS)])

    def ri(j, _):
        def rk(kk, _):
            row_idx[j, pl.ds(kk * L, L)] = j * CK + kk * L + iota16
            return 0
        return lax.fori_loop(0, CK // L, rk, 0)
    lax.fori_loop(0, N_PAD // CK, ri, 0)

    plsc.subcore_barrier()

    pltpu.sync_copy(ei_hbm.at[1, pl.ds(w * EPW, EPW)], dst_v)

    def body(i, _):
        for u in range(8):
            idx = dst_v[pl.ds((i * 8 + u) * L, L)]
            plsc.addupdate_scatter(hist_v, [idx], ones16)
        return 0
    lax.fori_loop(0, EPW // (8 * L), body, 0)
    for u in range(EPW // L - (EPW // (8 * L)) * 8):
        idx = dst_v[pl.ds(((EPW // (8 * L)) * 8 + u) * L, L)]
        plsc.addupdate_scatter(hist_v, [idx], ones16)

    def comb(j, _):
        pltpu.sync_copy(hist_v.at[pl.ds(j * CK, CK)],
                        acc.at[row_idx.at[j]], add=True)
        return 0
    lax.fori_loop(0, N_PAD // CK, comb, 0)

    plsc.subcore_barrier()
    pltpu.sync_copy(acc.at[pl.ds(s * (N_PAD // NS), N_PAD // NS)],
                    out_hbm.at[c, pl.ds(s * (N_PAD // NS), N_PAD // NS)])


@functools.cache
def _sc_scatter_kernel():
    mesh = plsc.VectorSubcoreMesh(core_axis_name="c", subcore_axis_name="s")
    return pl.kernel(
        _sc_scatter_body,
        out_type=jax.ShapeDtypeStruct((NC, N_PAD, D_H), jnp.float32),
        mesh=mesh,
        scratch_types=[
            pltpu.VMEM((EPW,), jnp.int32),
            pltpu.VMEM((EPW,), jnp.int32),
            pltpu.VMEM((2, K, D_H), jnp.float32),
            pltpu.VMEM((64, D_H), jnp.float32),
            pltpu.VMEM_SHARED((N_PAD, D_H), jnp.float32),
            pltpu.SemaphoreType.DMA,
            pltpu.SemaphoreType.DMA,
        ],
        compiler_params=pltpu.CompilerParams(
            needs_layout_passes=False, use_tc_tiling_on_sc=False),
    )


def _sc_scatter_body(g_hbm, ei_hbm, out_hbm,
                     src_v, dst_v, rows2, zbuf, acc, sem_g, sem_s):
    c = lax.axis_index("c")
    s = lax.axis_index("s")
    w = c * NS + s
    zeros16 = jnp.zeros((L,), jnp.float32)

    def zrow(i, _):
        def zcol(kk, _):
            zbuf[i, pl.ds(kk * L, L)] = zeros16
            return 0
        return lax.fori_loop(0, D_H // L, zcol, 0)
    lax.fori_loop(0, 64, zrow, 0)

    def zc(i, _):
        pltpu.sync_copy(zbuf, acc.at[pl.ds(s * RPT + i * 64, 64)])
        return 0
    lax.fori_loop(0, RPT // 64, zc, 0)

    plsc.subcore_barrier()

    pltpu.sync_copy(ei_hbm.at[0, pl.ds(w * EPW, EPW)], src_v)
    pltpu.sync_copy(ei_hbm.at[1, pl.ds(w * EPW, EPW)], dst_v)

    def gather(j, b):
        pltpu.async_copy(g_hbm.at[src_v.at[pl.ds(j * K, K)]],
                         rows2.at[b], sem_g).wait()

    def scat_start(j, b):
        pltpu.async_copy(rows2.at[b], acc.at[dst_v.at[pl.ds(j * K, K)]],
                         sem_s, add=True)

    def scat_wait(j, b):
        pltpu.make_async_copy(rows2.at[b],
                              acc.at[dst_v.at[pl.ds(j * K, K)]],
                              sem_s).wait()

    gather(0, 0)

    def body(jj, _):
        j = 2 * jj
        scat_start(j, 0)
        gather(j + 1, 1)
        scat_wait(j, 0)
        scat_start(j + 1, 1)
        gather(j + 2, 0)
        scat_wait(j + 1, 1)
        return 0
    lax.fori_loop(0, (T_CH - 2) // 2, body, 0)

    scat_start(T_CH - 2, 0)
    gather(T_CH - 1, 1)
    scat_wait(T_CH - 2, 0)
    scat_start(T_CH - 1, 1)
    pltpu.async_copy(g_hbm.at[src_v.at[pl.ds(T_CH * K, TAIL)]],
                     rows2.at[0].at[pl.ds(0, TAIL)], sem_g).wait()
    scat_wait(T_CH - 1, 1)
    pltpu.sync_copy(rows2.at[0].at[pl.ds(0, TAIL)],
                    acc.at[dst_v.at[pl.ds(T_CH * K, TAIL)]], add=True)

    plsc.subcore_barrier()
    pltpu.sync_copy(acc.at[pl.ds(s * RPT, RPT)],
                    out_hbm.at[c, pl.ds(s * RPT, RPT)])


_R = 2048
_G = -(-N // _R)


def _dis_col(d_ref):
    deg = d_ref[0:1, :] + d_ref[1:2, :] + 1.0
    dis = lax.rsqrt(deg)
    return jnp.reshape(dis, (_R, 1))


def _tc1a_body(x_ref, w_ref, h_ref):
    h_ref[...] = jnp.dot(x_ref[...], w_ref[...],
                         preferred_element_type=jnp.float32)


def _tc1a(x, W1):
    return pl.pallas_call(
        _tc1a_body,
        grid=(_G,),
        in_specs=[
            pl.BlockSpec((_R, D_IN), lambda m: (m, 0)),
            pl.BlockSpec((D_IN, D_H), lambda m: (0, 0)),
        ],
        out_specs=pl.BlockSpec((_R, D_H), lambda m: (m, 0)),
        out_shape=jax.ShapeDtypeStruct((N, D_H), jnp.float32),
    )(x, W1)


def _tc1b_body(h_ref, d_ref, g_ref):
    g_ref[...] = h_ref[...] * _dis_col(d_ref)


def _tc1b(h, deg):
    return pl.pallas_call(
        _tc1b_body,
        grid=(_G,),
        in_specs=[
            pl.BlockSpec((_R, D_H), lambda m: (m, 0)),
            pl.BlockSpec((NC, _R), lambda m: (0, m)),
        ],
        out_specs=pl.BlockSpec((_R, D_H), lambda m: (m, 0)),
        out_shape=jax.ShapeDtypeStruct((N, D_H), jnp.float32),
    )(h, deg)


def _tc2_body(p_ref, g1_ref, d_ref, b1_ref, w2_ref, g2_ref):
    dis = _dis_col(d_ref)
    agg = p_ref[0] + p_ref[1] + g1_ref[...]
    h = jnp.maximum(agg * dis + b1_ref[0:1, :], 0.0)
    g2_ref[...] = jnp.dot(h, w2_ref[...],
                          preferred_element_type=jnp.float32) * dis


def _tc2(p, g1, deg, b1r, W2):
    return pl.pallas_call(
        _tc2_body,
        grid=(_G,),
        in_specs=[
            pl.BlockSpec((NC, _R, D_H), lambda m: (0, m, 0)),
            pl.BlockSpec((_R, D_H), lambda m: (m, 0)),
            pl.BlockSpec((NC, _R), lambda m: (0, m)),
            pl.BlockSpec((8, D_H), lambda m: (0, 0)),
            pl.BlockSpec((D_H, D_H), lambda m: (0, 0)),
        ],
        out_specs=pl.BlockSpec((_R, D_H), lambda m: (m, 0)),
        out_shape=jax.ShapeDtypeStruct((N, D_H), jnp.float32),
    )(p, g1, deg, b1r, W2)


def _tc3_body(p_ref, g2_ref, d_ref, b2_ref, wc_ref, bc_ref, y_ref):
    dis = _dis_col(d_ref)
    agg = p_ref[0] + p_ref[1] + g2_ref[...]
    h = jnp.maximum(agg * dis + b2_ref[0:1, :], 0.0)
    z = jnp.dot(h, wc_ref[...],
                preferred_element_type=jnp.float32) + bc_ref[0:1, 0:1]
    y_ref[...] = jax.nn.sigmoid(z)


def _tc3(p, g2, deg, b2r, Wc, bcr):
    return pl.pallas_call(
        _tc3_body,
        grid=(_G,),
        in_specs=[
            pl.BlockSpec((NC, _R, D_H), lambda m: (0, m, 0)),
            pl.BlockSpec((_R, D_H), lambda m: (m, 0)),
            pl.BlockSpec((NC, _R), lambda m: (0, m)),
            pl.BlockSpec((8, D_H), lambda m: (0, 0)),
            pl.BlockSpec((D_H, 1), lambda m: (0, 0)),
            pl.BlockSpec((8, 1), lambda m: (0, 0)),
        ],
        out_specs=pl.BlockSpec((_R, 1), lambda m: (m, 0)),
        out_shape=jax.ShapeDtypeStruct((N, 1), jnp.float32),
    )(p, g2, deg, b2r, Wc, bcr)


def kernel(x, edge_index, W1, b1, W2, b2, Wc, bc):
    ei = edge_index.astype(jnp.int32)

    h1 = _tc1a(x, W1)
    deg = _sc_degree_kernel()(ei)
    g1 = _tc1b(h1, deg)

    b1r = jnp.broadcast_to(b1[None, :], (8, D_H))
    b2r = jnp.broadcast_to(b2[None, :], (8, D_H))
    bcr = jnp.broadcast_to(bc[None, :], (8, 1))

    p = _sc_scatter_kernel()(g1, ei)
    g2 = _tc2(p, g1, deg, b1r, W2)

    q = _sc_scatter_kernel()(g2, ei)
    y = _tc3(q, g2, deg, b2r, Wc, bcr)
    return y
